# baseline (speedup 1.0000x reference)
import functools

import jax
import jax.numpy as jnp
from jax import lax
from jax.experimental import pallas as pl
from jax.experimental.pallas import tpu as pltpu
from jax.experimental.pallas import tpu_sc as plsc

N_VOCAB = 100000
D_MODEL = 128
BATCH = 4
SEQ = 8192

NUM_CORES = 2
NUM_SUBCORES = 16
NUM_WORKERS = NUM_CORES * NUM_SUBCORES
S_PER_W = SEQ // NUM_WORKERS
SUB = 64
K = (BATCH * S_PER_W) // SUB
SPB = K // BATCH
LANES = 16
NBUF = 6

_mesh = plsc.VectorSubcoreMesh(core_axis_name="c", subcore_axis_name="s")


@functools.partial(
    pl.kernel,
    mesh=_mesh,
    out_type=jax.ShapeDtypeStruct((BATCH, SEQ, D_MODEL), jnp.float32),
    scratch_types=[
        pltpu.VMEM((K, SUB), jnp.int32),
        pltpu.VMEM((S_PER_W, D_MODEL), jnp.float32),
    ] + [pltpu.VMEM((SUB, D_MODEL), jnp.float32) for _ in range(NBUF)]
      + [pltpu.SemaphoreType.DMA for _ in range(2 * NBUF + 2)],
)
def _emb_kernel(x_hbm, emb_hbm, pos_hbm, out_hbm, idx_v, pos_v, *rest):
    bufs = rest[:NBUF]
    gsems = rest[NBUF:2 * NBUF]
    osems = rest[2 * NBUF:3 * NBUF]
    psem = rest[3 * NBUF]
    isem = rest[3 * NBUF + 1]

    wid = lax.axis_index("s") * NUM_CORES + lax.axis_index("c")
    s0 = wid * S_PER_W

    pos_copy = pltpu.async_copy(pos_hbm.at[pl.ds(s0, S_PER_W)], pos_v, psem)

    idx_copies = []
    for k in range(K):
        idx_copies.append(pltpu.async_copy(
            x_hbm.at[k // SPB, pl.ds(s0 + (k % SPB) * SUB, SUB)], idx_v.at[k], isem))
    for c in idx_copies:
        c.wait()

    gathers = [None] * NBUF
    outs = [None] * NBUF
    for k in range(NBUF - 1):
        gathers[k] = pltpu.async_copy(
            emb_hbm.at[idx_v.at[k]], bufs[k], gsems[k])

    pos_copy.wait()

    for k in range(K):
        b = k % NBUF
        if k + NBUF - 1 < K:
            nb = (k + NBUF - 1) % NBUF
            if outs[nb] is not None:
                outs[nb].wait()
            gathers[nb] = pltpu.async_copy(
                emb_hbm.at[idx_v.at[k + NBUF - 1]], bufs[nb], gsems[nb])

        cur = bufs[b]
        gathers[b].wait()

        jb = (k % SPB) * SUB

        def add_pos(r, carry, cur=cur, jb=jb):
            for cc in range(D_MODEL // LANES):
                c = cc * LANES
                cur[r, pl.ds(c, LANES)] = (
                    cur[r, pl.ds(c, LANES)] + pos_v[jb + r, pl.ds(c, LANES)])
            return carry

        lax.fori_loop(0, SUB, add_pos, 0)

        outs[b] = pltpu.async_copy(
            cur, out_hbm.at[k // SPB, pl.ds(s0 + (k % SPB) * SUB, SUB)], osems[b])

    for b in range(NBUF):
        if outs[b] is not None:
            outs[b].wait()


def kernel(x, emb, pos_emb):
    return _emb_kernel(x.astype(jnp.int32), emb, pos_emb)


# speedup vs baseline: 1.0323x; 1.0323x over previous
import functools

import jax
import jax.numpy as jnp
from jax import lax
from jax.experimental import pallas as pl
from jax.experimental.pallas import tpu as pltpu
from jax.experimental.pallas import tpu_sc as plsc

N_VOCAB = 100000
D_MODEL = 128
BATCH = 4
SEQ = 8192

NUM_CORES = 2
NUM_SUBCORES = 16
NUM_WORKERS = NUM_CORES * NUM_SUBCORES
S_PER_W = SEQ // NUM_WORKERS
SUB = 128
K = (BATCH * S_PER_W) // SUB
SPB = K // BATCH
LANES = 16
NBUF = 4

_mesh = plsc.VectorSubcoreMesh(core_axis_name="c", subcore_axis_name="s")


@functools.partial(
    pl.kernel,
    mesh=_mesh,
    out_type=jax.ShapeDtypeStruct((BATCH, SEQ, D_MODEL), jnp.float32),
    scratch_types=[
        pltpu.VMEM((K, SUB), jnp.int32),
        pltpu.VMEM((S_PER_W, D_MODEL), jnp.float32),
    ] + [pltpu.VMEM((SUB, D_MODEL), jnp.float32) for _ in range(NBUF)]
      + [pltpu.SemaphoreType.DMA for _ in range(2 * NBUF + 2)],
)
def _emb_kernel(x_hbm, emb_hbm, pos_hbm, out_hbm, idx_v, pos_v, *rest):
    bufs = rest[:NBUF]
    gsems = rest[NBUF:2 * NBUF]
    osems = rest[2 * NBUF:3 * NBUF]
    psem = rest[3 * NBUF]
    isem = rest[3 * NBUF + 1]

    wid = lax.axis_index("s") * NUM_CORES + lax.axis_index("c")
    s0 = wid * S_PER_W

    def stage_idx(k, sem):
        return pltpu.async_copy(
            x_hbm.at[k // SPB, pl.ds(s0 + (k % SPB) * SUB, SUB)],
            idx_v.at[k], sem)

    early = [stage_idx(k, osems[k]) for k in range(NBUF - 1)]
    late = [stage_idx(k, isem) for k in range(NBUF - 1, K)]

    pos_copy = pltpu.async_copy(pos_hbm.at[pl.ds(s0, S_PER_W)], pos_v, psem)

    gathers = [None] * NBUF
    outs = [None] * NBUF
    for k in range(NBUF - 1):
        early[k].wait()
        gathers[k] = pltpu.async_copy(
            emb_hbm.at[idx_v.at[k]], bufs[k], gsems[k])

    for c in late:
        c.wait()
    pos_copy.wait()

    for k in range(K):
        b = k % NBUF
        if k + NBUF - 1 < K:
            nb = (k + NBUF - 1) % NBUF
            if outs[nb] is not None:
                outs[nb].wait()
            gathers[nb] = pltpu.async_copy(
                emb_hbm.at[idx_v.at[k + NBUF - 1]], bufs[nb], gsems[nb])

        cur = bufs[b]
        gathers[b].wait()

        jb = (k % SPB) * SUB

        def add_pos(r, carry, cur=cur, jb=jb):
            for cc in range(D_MODEL // LANES):
                c = cc * LANES
                cur[r, pl.ds(c, LANES)] = (
                    cur[r, pl.ds(c, LANES)] + pos_v[jb + r, pl.ds(c, LANES)])
            return carry

        lax.fori_loop(0, SUB, add_pos, 0)

        outs[b] = pltpu.async_copy(
            cur, out_hbm.at[k // SPB, pl.ds(s0 + (k % SPB) * SUB, SUB)], osems[b])

    for b in range(NBUF):
        if outs[b] is not None:
            outs[b].wait()


def kernel(x, emb, pos_emb):
    return _emb_kernel(x.astype(jnp.int32), emb, pos_emb)


# speedup vs baseline: 1.0485x; 1.0157x over previous
import functools

import jax
import jax.numpy as jnp
from jax import lax
from jax.experimental import pallas as pl
from jax.experimental.pallas import tpu as pltpu
from jax.experimental.pallas import tpu_sc as plsc

N_VOCAB = 100000
D_MODEL = 128
BATCH = 4
SEQ = 8192

NUM_CORES = 2
NUM_SUBCORES = 16
NUM_WORKERS = NUM_CORES * NUM_SUBCORES
S_PER_W = SEQ // NUM_WORKERS
SUB = 128
GPS = S_PER_W // SUB
LANES = 16
NBUF = 2

_mesh = plsc.VectorSubcoreMesh(core_axis_name="c", subcore_axis_name="s")


@functools.partial(
    pl.kernel,
    mesh=_mesh,
    out_type=jax.ShapeDtypeStruct((BATCH, SEQ, D_MODEL), jnp.float32),
    scratch_types=[
        pltpu.VMEM((BATCH * GPS, SUB), jnp.int32),
        pltpu.VMEM((S_PER_W, D_MODEL), jnp.float32),
    ] + [pltpu.VMEM((S_PER_W, D_MODEL), jnp.float32) for _ in range(NBUF)]
      + [pltpu.SemaphoreType.DMA for _ in range(2 * NBUF + 2)],
)
def _emb_kernel(x_hbm, emb_hbm, pos_hbm, out_hbm, idx_v, pos_v, *rest):
    bufs = rest[:NBUF]
    gsems = rest[NBUF:2 * NBUF]
    osems = rest[2 * NBUF:3 * NBUF]
    psem = rest[3 * NBUF]
    isem = rest[3 * NBUF + 1]

    wid = lax.axis_index("s") * NUM_CORES + lax.axis_index("c")
    s0 = wid * S_PER_W

    def stage_idx(b, g, sem):
        return pltpu.async_copy(
            x_hbm.at[b, pl.ds(s0 + g * SUB, SUB)], idx_v.at[b * GPS + g], sem)

    early = [stage_idx(0, g, osems[0]) for g in range(GPS)]
    late = [stage_idx(b, g, isem) for b in range(1, BATCH) for g in range(GPS)]

    pos_copy = pltpu.async_copy(pos_hbm.at[pl.ds(s0, S_PER_W)], pos_v, psem)

    def gather_step(b, buf, sem):
        return [pltpu.async_copy(
            emb_hbm.at[idx_v.at[b * GPS + g]],
            buf.at[pl.ds(g * SUB, SUB)], sem) for g in range(GPS)]

    for c in early:
        c.wait()
    gathers = [None] * NBUF
    outs = [None] * NBUF
    gathers[0] = gather_step(0, bufs[0], gsems[0])

    for c in late:
        c.wait()
    pos_copy.wait()

    for b in range(BATCH):
        s = b % NBUF
        if b + 1 < BATCH:
            ns = (b + 1) % NBUF
            if outs[ns] is not None:
                outs[ns].wait()
            gathers[ns] = gather_step(b + 1, bufs[ns], gsems[ns])

        cur = bufs[s]
        for g in gathers[s]:
            g.wait()

        def add_pos(r, carry, cur=cur):
            for cc in range(D_MODEL // LANES):
                c = cc * LANES
                cur[r, pl.ds(c, LANES)] = (
                    cur[r, pl.ds(c, LANES)] + pos_v[r, pl.ds(c, LANES)])
            return carry

        lax.fori_loop(0, S_PER_W, add_pos, 0)

        outs[s] = pltpu.async_copy(
            cur, out_hbm.at[b, pl.ds(s0, S_PER_W)], osems[s])

    for s in range(NBUF):
        if outs[s] is not None:
            outs[s].wait()


def kernel(x, emb, pos_emb):
    return _emb_kernel(x.astype(jnp.int32), emb, pos_emb)


# speedup vs baseline: 1.0724x; 1.0228x over previous
import functools

import jax
import jax.numpy as jnp
from jax import lax
from jax.experimental import pallas as pl
from jax.experimental.pallas import tpu as pltpu
from jax.experimental.pallas import tpu_sc as plsc

N_VOCAB = 100000
D_MODEL = 128
BATCH = 4
SEQ = 8192

NUM_CORES = 2
NUM_SUBCORES = 16
NUM_WORKERS = NUM_CORES * NUM_SUBCORES
S_PER_W = SEQ // NUM_WORKERS
SUB = 128
GPS = S_PER_W // SUB
LANES = 16
NBUF = 2

_mesh = plsc.VectorSubcoreMesh(core_axis_name="c", subcore_axis_name="s")


@functools.partial(
    pl.kernel,
    mesh=_mesh,
    out_type=jax.ShapeDtypeStruct((BATCH, SEQ, D_MODEL), jnp.float32),
    scratch_types=[
        pltpu.VMEM((BATCH * GPS, SUB), jnp.int32),
        pltpu.VMEM((S_PER_W, D_MODEL), jnp.float32),
    ] + [pltpu.VMEM((S_PER_W, D_MODEL), jnp.float32) for _ in range(NBUF)]
      + [pltpu.SemaphoreType.DMA for _ in range(2 * NBUF * GPS + GPS + 1)],
)
def _emb_kernel(x_hbm, emb_hbm, pos_hbm, out_hbm, idx_v, pos_v, *rest):
    bufs = rest[:NBUF]
    sems = rest[NBUF:]
    gsems = [[sems[s * GPS + g] for g in range(GPS)] for s in range(NBUF)]
    osems = [[sems[NBUF * GPS + s * GPS + g] for g in range(GPS)]
             for s in range(NBUF)]
    psems = [sems[2 * NBUF * GPS + g] for g in range(GPS)]
    isem = sems[2 * NBUF * GPS + GPS]

    wid = lax.axis_index("s") * NUM_CORES + lax.axis_index("c")
    s0 = wid * S_PER_W

    def stage_idx(b, g, sem):
        return pltpu.async_copy(
            x_hbm.at[b, pl.ds(s0 + g * SUB, SUB)], idx_v.at[b * GPS + g], sem)

    early = [stage_idx(0, g, osems[0][g]) for g in range(GPS)]
    late = [stage_idx(b, g, isem) for b in range(1, BATCH) for g in range(GPS)]

    def gather_half(b, s, g):
        return pltpu.async_copy(
            emb_hbm.at[idx_v.at[b * GPS + g]],
            bufs[s].at[pl.ds(g * SUB, SUB)], gsems[s][g])

    gathers = [[None] * GPS for _ in range(NBUF)]
    outs = [[None] * GPS for _ in range(NBUF)]
    for g in range(GPS):
        early[g].wait()
        gathers[0][g] = gather_half(0, 0, g)

    pos_copies = [pltpu.async_copy(
        pos_hbm.at[pl.ds(s0 + g * SUB, SUB)],
        pos_v.at[pl.ds(g * SUB, SUB)], psems[g]) for g in range(GPS)]
    pos_ready = [False] * GPS

    for c in late:
        c.wait()

    final_outs = []
    for b in range(BATCH):
        s = b % NBUF
        ns = (b + 1) % NBUF
        cur = bufs[s]
        for g in range(GPS):
            if b + 1 < BATCH:
                if outs[ns][g] is not None:
                    outs[ns][g].wait()
                    outs[ns][g] = None
                gathers[ns][g] = gather_half(b + 1, ns, g)

            gathers[s][g].wait()
            if not pos_ready[g]:
                pos_copies[g].wait()
                pos_ready[g] = True

            def add_rows(lo, n, cur=cur):
                def body(r, carry):
                    for cc in range(D_MODEL // LANES):
                        c = cc * LANES
                        cur[lo + r, pl.ds(c, LANES)] = (
                            cur[lo + r, pl.ds(c, LANES)]
                            + pos_v[lo + r, pl.ds(c, LANES)])
                    return carry
                lax.fori_loop(0, n, body, 0)

            if b == BATCH - 1 and g == GPS - 1:
                q = SUB // 2
                for i in range(2):
                    add_rows(g * SUB + i * q, q)
                    final_outs.append(pltpu.async_copy(
                        cur.at[pl.ds(g * SUB + i * q, q)],
                        out_hbm.at[b, pl.ds(s0 + g * SUB + i * q, q)],
                        psems[i]))
            else:
                add_rows(g * SUB, SUB)
                outs[s][g] = pltpu.async_copy(
                    cur.at[pl.ds(g * SUB, SUB)],
                    out_hbm.at[b, pl.ds(s0 + g * SUB, SUB)], osems[s][g])

    drain = final_outs + [outs[s][g] for s in range(NBUF) for g in range(GPS)]
    for o in drain:
        if o is not None:
            o.wait()


def kernel(x, emb, pos_emb):
    return _emb_kernel(x.astype(jnp.int32), emb, pos_emb)


# speedup vs baseline: 1.0812x; 1.0081x over previous
import functools

import jax
import jax.numpy as jnp
from jax import lax
from jax.experimental import pallas as pl
from jax.experimental.pallas import tpu as pltpu
from jax.experimental.pallas import tpu_sc as plsc

N_VOCAB = 100000
D_MODEL = 128
BATCH = 4
SEQ = 8192

NUM_CORES = 2
NUM_SUBCORES = 16
NUM_WORKERS = NUM_CORES * NUM_SUBCORES
S_PER_W = SEQ // NUM_WORKERS
SUB = 128
GPS = S_PER_W // SUB
LANES = 16
NBUF = 2

_mesh = plsc.VectorSubcoreMesh(core_axis_name="c", subcore_axis_name="s")


@functools.partial(
    pl.kernel,
    mesh=_mesh,
    out_type=jax.ShapeDtypeStruct((BATCH, SEQ, D_MODEL), jnp.float32),
    scratch_types=[
        pltpu.VMEM((BATCH * GPS, SUB), jnp.int32),
        pltpu.VMEM((S_PER_W, D_MODEL), jnp.float32),
    ] + [pltpu.VMEM((S_PER_W, D_MODEL), jnp.float32) for _ in range(NBUF)]
      + [pltpu.SemaphoreType.DMA for _ in range(2 * NBUF * GPS + GPS + 1)],
)
def _emb_kernel(x_hbm, emb_hbm, pos_hbm, out_hbm, idx_v, pos_v, *rest):
    bufs = rest[:NBUF]
    sems = rest[NBUF:]
    gsems = [[sems[s * GPS + g] for g in range(GPS)] for s in range(NBUF)]
    osems = [[sems[NBUF * GPS + s * GPS + g] for g in range(GPS)]
             for s in range(NBUF)]
    psems = [sems[2 * NBUF * GPS + g] for g in range(GPS)]
    isem = sems[2 * NBUF * GPS + GPS]

    wid = lax.axis_index("s") * NUM_CORES + lax.axis_index("c")
    s0 = wid * S_PER_W

    def stage_idx(b, g, sem):
        return pltpu.async_copy(
            x_hbm.at[b, pl.ds(s0 + g * SUB, SUB)], idx_v.at[b * GPS + g], sem)

    early = [stage_idx(0, g, osems[0][g]) for g in range(GPS)]
    late = [stage_idx(b, g, isem) for b in range(1, BATCH) for g in range(GPS)]

    pos_copies = [pltpu.async_copy(
        pos_hbm.at[pl.ds(s0 + g * SUB, SUB)],
        pos_v.at[pl.ds(g * SUB, SUB)], psems[g]) for g in range(GPS)]
    pos_ready = [False] * GPS

    def gather_half(b, s, g):
        return pltpu.async_copy(
            emb_hbm.at[idx_v.at[b * GPS + g]],
            bufs[s].at[pl.ds(g * SUB, SUB)], gsems[s][g])

    gathers = [[None] * GPS for _ in range(NBUF)]
    outs = [[None] * GPS for _ in range(NBUF)]
    for g in range(GPS):
        early[g].wait()
        gathers[0][g] = gather_half(0, 0, g)

    for c in late:
        c.wait()

    final_outs = []
    for b in range(BATCH):
        s = b % NBUF
        ns = (b + 1) % NBUF
        cur = bufs[s]
        for g in range(GPS):
            if b + 1 < BATCH:
                if outs[ns][g] is not None:
                    outs[ns][g].wait()
                    outs[ns][g] = None
                gathers[ns][g] = gather_half(b + 1, ns, g)

            gathers[s][g].wait()
            if not pos_ready[g]:
                pos_copies[g].wait()
                pos_ready[g] = True

            def add_rows(lo, n, cur=cur):
                def body(r, carry):
                    for cc in range(D_MODEL // LANES):
                        c = cc * LANES
                        cur[lo + r, pl.ds(c, LANES)] = (
                            cur[lo + r, pl.ds(c, LANES)]
                            + pos_v[lo + r, pl.ds(c, LANES)])
                    return carry
                lax.fori_loop(0, n, body, 0)

            if b == BATCH - 1 and g == GPS - 1:
                q = SUB // 2
                for i in range(2):
                    add_rows(g * SUB + i * q, q)
                    final_outs.append(pltpu.async_copy(
                        cur.at[pl.ds(g * SUB + i * q, q)],
                        out_hbm.at[b, pl.ds(s0 + g * SUB + i * q, q)],
                        psems[i]))
            else:
                add_rows(g * SUB, SUB)
                outs[s][g] = pltpu.async_copy(
                    cur.at[pl.ds(g * SUB, SUB)],
                    out_hbm.at[b, pl.ds(s0 + g * SUB, SUB)], osems[s][g])

    drain = final_outs + [outs[s][g] for s in range(NBUF) for g in range(GPS)]
    for o in drain:
        if o is not None:
            o.wait()


def kernel(x, emb, pos_emb):
    return _emb_kernel(x.astype(jnp.int32), emb, pos_emb)


# speedup vs baseline: 1.0910x; 1.0091x over previous
import functools

import jax
import jax.numpy as jnp
from jax import lax
from jax.experimental import pallas as pl
from jax.experimental.pallas import tpu as pltpu
from jax.experimental.pallas import tpu_sc as plsc

N_VOCAB = 100000
D_MODEL = 128
BATCH = 4
SEQ = 8192

NUM_CORES = 2
NUM_SUBCORES = 16
NUM_WORKERS = NUM_CORES * NUM_SUBCORES
S_PER_W = SEQ // NUM_WORKERS
SUB = 128
GPS = S_PER_W // SUB
LANES = 16
NBUF = 2

_mesh = plsc.VectorSubcoreMesh(core_axis_name="c", subcore_axis_name="s")


@functools.partial(
    pl.kernel,
    mesh=_mesh,
    out_type=jax.ShapeDtypeStruct((BATCH, SEQ, D_MODEL), jnp.float32),
    scratch_types=[
        pltpu.VMEM((BATCH * GPS, SUB), jnp.int32),
        pltpu.VMEM((S_PER_W, D_MODEL), jnp.float32),
    ] + [pltpu.VMEM((S_PER_W, D_MODEL), jnp.float32) for _ in range(NBUF)]
      + [pltpu.SemaphoreType.DMA for _ in range(2 * NBUF * GPS + 2)],
)
def _emb_kernel(x_hbm, emb_hbm, pos_hbm, out_hbm, idx_v, pos_v, *rest):
    bufs = rest[:NBUF]
    sems = rest[NBUF:]
    gsems = [[sems[s * GPS + g] for g in range(GPS)] for s in range(NBUF)]
    osems = [[sems[NBUF * GPS + s * GPS + g] for g in range(GPS)]
             for s in range(NBUF)]
    psem = sems[2 * NBUF * GPS]
    isem = sems[2 * NBUF * GPS + 1]

    wid = lax.axis_index("s") * NUM_CORES + lax.axis_index("c")
    s0 = wid * S_PER_W

    def stage_idx(b, g, sem):
        return pltpu.async_copy(
            x_hbm.at[b, pl.ds(s0 + g * SUB, SUB)], idx_v.at[b * GPS + g], sem)

    early = [stage_idx(0, g, osems[0][g]) for g in range(GPS)]
    late = [stage_idx(b, g, isem) for b in range(1, BATCH) for g in range(GPS)]

    pos_copy = pltpu.async_copy(pos_hbm.at[pl.ds(s0, S_PER_W)], pos_v, psem)

    def gather_half(b, s, g):
        return pltpu.async_copy(
            emb_hbm.at[idx_v.at[b * GPS + g]],
            bufs[s].at[pl.ds(g * SUB, SUB)], gsems[s][g])

    gathers = [[None] * GPS for _ in range(NBUF)]
    outs = [[None] * GPS for _ in range(NBUF)]
    for g in range(GPS):
        early[g].wait()
        gathers[0][g] = gather_half(0, 0, g)

    for c in late:
        c.wait()
    pos_copy.wait()

    for b in range(BATCH):
        s = b % NBUF
        ns = (b + 1) % NBUF
        cur = bufs[s]
        for g in range(GPS):
            if b + 1 < BATCH:
                if outs[ns][g] is not None:
                    outs[ns][g].wait()
                gathers[ns][g] = gather_half(b + 1, ns, g)

            gathers[s][g].wait()

            def add_pos(r, carry, cur=cur, base=g * SUB):
                for cc in range(D_MODEL // LANES):
                    c = cc * LANES
                    cur[base + r, pl.ds(c, LANES)] = (
                        cur[base + r, pl.ds(c, LANES)]
                        + pos_v[base + r, pl.ds(c, LANES)])
                return carry

            lax.fori_loop(0, SUB, add_pos, 0)

            outs[s][g] = pltpu.async_copy(
                cur.at[pl.ds(g * SUB, SUB)],
                out_hbm.at[b, pl.ds(s0 + g * SUB, SUB)], osems[s][g])

    for s in range(NBUF):
        for g in range(GPS):
            if outs[s][g] is not None:
                outs[s][g].wait()


def kernel(x, emb, pos_emb):
    return _emb_kernel(x.astype(jnp.int32), emb, pos_emb)


# speedup vs baseline: 1.1055x; 1.0134x over previous
import functools

import jax
import jax.numpy as jnp
from jax import lax
from jax.experimental import pallas as pl
from jax.experimental.pallas import tpu as pltpu
from jax.experimental.pallas import tpu_sc as plsc

N_VOCAB = 100000
D_MODEL = 128
BATCH = 4
SEQ = 8192

NUM_CORES = 2
NUM_SUBCORES = 16
NUM_WORKERS = NUM_CORES * NUM_SUBCORES
S_PER_W = SEQ // NUM_WORKERS
SUB = 128
SPB = S_PER_W // SUB
K = BATCH * SPB
LANES = 16
RING = 5
PRIME = 3

_mesh = plsc.VectorSubcoreMesh(core_axis_name="c", subcore_axis_name="s")


@functools.partial(
    pl.kernel,
    mesh=_mesh,
    out_type=jax.ShapeDtypeStruct((BATCH, SEQ, D_MODEL), jnp.float32),
    scratch_types=[
        pltpu.VMEM((K, SUB), jnp.int32),
        pltpu.VMEM((S_PER_W, D_MODEL), jnp.float32),
    ] + [pltpu.VMEM((SUB, D_MODEL), jnp.float32) for _ in range(RING)]
      + [pltpu.SemaphoreType.DMA for _ in range(2 * RING + 2)],
)
def _emb_kernel(x_hbm, emb_hbm, pos_hbm, out_hbm, idx_v, pos_v, *rest):
    bufs = rest[:RING]
    sems = rest[RING:]
    gsems = sems[:RING]
    osems = sems[RING:2 * RING]
    psem = sems[2 * RING]
    isem = sems[2 * RING + 1]

    wid = lax.axis_index("s") * NUM_CORES + lax.axis_index("c")
    s0 = wid * S_PER_W

    def stage_idx(k, sem):
        return pltpu.async_copy(
            x_hbm.at[k // SPB, pl.ds(s0 + (k % SPB) * SUB, SUB)],
            idx_v.at[k], sem)

    early = [stage_idx(k, osems[k]) for k in range(PRIME)]
    late = [stage_idx(k, isem) for k in range(PRIME, K)]

    pos_copy = pltpu.async_copy(pos_hbm.at[pl.ds(s0, S_PER_W)], pos_v, psem)

    def gather_block(k):
        return pltpu.async_copy(
            emb_hbm.at[idx_v.at[k]], bufs[k % RING], gsems[k % RING])

    gathers = [None] * RING
    outs = [None] * RING
    for k in range(PRIME):
        early[k].wait()
        gathers[k] = gather_block(k)

    for c in late:
        c.wait()
    pos_copy.wait()

    for k in range(K):
        sl = k % RING
        kp = k + PRIME
        if kp < K:
            ps = kp % RING
            if outs[ps] is not None:
                outs[ps].wait()
                outs[ps] = None
            gathers[ps] = gather_block(kp)

        cur = bufs[sl]
        gathers[sl].wait()

        def add_pos(r, carry, cur=cur, base=(k % SPB) * SUB):
            for cc in range(D_MODEL // LANES):
                c = cc * LANES
                cur[r, pl.ds(c, LANES)] = (
                    cur[r, pl.ds(c, LANES)] + pos_v[base + r, pl.ds(c, LANES)])
            return carry

        lax.fori_loop(0, SUB, add_pos, 0)

        outs[sl] = pltpu.async_copy(
            cur, out_hbm.at[k // SPB, pl.ds(s0 + (k % SPB) * SUB, SUB)],
            osems[sl])

    for sl in range(RING):
        if outs[sl] is not None:
            outs[sl].wait()


def kernel(x, emb, pos_emb):
    return _emb_kernel(x.astype(jnp.int32), emb, pos_emb)
